# baseline (speedup 1.0000x reference)
import functools

import jax
import jax.numpy as jnp
import numpy as np
from jax import lax
from jax.experimental import pallas as pl
from jax.experimental.pallas import tpu as pltpu
from jax.experimental.pallas import tpu_sc as plsc

N = 524288
N_LEVELS = 16
F = 2
LOG2_T = 19
TABLE_SIZE = 1 << LOG2_T
BASE_RES = 16
SCALE = 1.38
WIDTH = 64
OUT_DIM = 16
IN_DIM = N_LEVELS * F

NC = 2
NS = 16
L = 16
NW = NC * NS
PW = N // NW
B = 64
CH = PW // B
RPP = N_LEVELS * 8
R = B * RPP

P1 = 2654435761 - (1 << 32)
P2 = 805459861
MASK = TABLE_SIZE - 1

RES = [float(np.floor(BASE_RES * SCALE**l)) for l in range(N_LEVELS)]

_mesh = plsc.VectorSubcoreMesh(core_axis_name="c", subcore_axis_name="s")


@functools.partial(
    pl.kernel,
    out_type=jax.ShapeDtypeStruct((IN_DIM, N), jnp.float32),
    mesh=_mesh,
    compiler_params=pltpu.CompilerParams(
        use_tc_tiling_on_sc=False, needs_layout_passes=False
    ),
    scratch_types=[
        pltpu.VMEM((B * 3,), jnp.float32),
        pltpu.VMEM((B * 3,), jnp.float32),
        pltpu.VMEM((2 * R,), jnp.int32),
        pltpu.VMEM((2 * R,), jnp.int32),
        pltpu.VMEM((R,), jnp.float32),
        pltpu.VMEM((R,), jnp.float32),
        pltpu.VMEM((R,), jnp.float32),
        pltpu.VMEM((R,), jnp.float32),
        pltpu.VMEM((IN_DIM, B), jnp.float32),
        pltpu.SemaphoreType.DMA,
        pltpu.SemaphoreType.DMA,
        pltpu.SemaphoreType.DMA,
        pltpu.SemaphoreType.DMA,
    ],
)
def _encode(
    xyz, tab0, enc,
    xyzv0, xyzv1, idx0, idx1, r0e, r0o, r1e, r1o, outv,
    s0e, s0o, s1e, s1o,
):
    wid = lax.axis_index("s") * NC + lax.axis_index("c")
    iota = lax.iota(jnp.int32, L)

    def stage_and_index(i, xyzv, idxv):
        base = wid * PW + i * B
        pltpu.sync_copy(xyz.at[pl.ds(base * 3, B * 3)], xyzv)

        def phase_a(v, c2):
            pids3 = (v * L + iota) * 3
            x = plsc.load_gather(xyzv, [pids3])
            y = plsc.load_gather(xyzv, [pids3 + 1])
            z = plsc.load_gather(xyzv, [pids3 + 2])
            for l in range(N_LEVELS):
                res = RES[l]
                px = (x * res).astype(jnp.int32)
                py = (y * res).astype(jnp.int32)
                pz = (z * res).astype(jnp.int32)
                loff = l * TABLE_SIZE
                hyt = py * P1
                hy = (jnp.bitwise_and(hyt, MASK), jnp.bitwise_and(hyt + P1, MASK))
                hzt = pz * P2
                hz = (
                    jnp.bitwise_or(jnp.bitwise_and(hzt, MASK), loff),
                    jnp.bitwise_or(jnp.bitwise_and(hzt + P2, MASK), loff),
                )
                hxy = (px ^ hy[0], px ^ hy[1], (px + 1) ^ hy[0], (px + 1) ^ hy[1])
                for c in range(8):
                    bx, by, bz = (c >> 2) & 1, (c >> 1) & 1, c & 1
                    h = hxy[bx * 2 + by] ^ hz[bz]
                    h2 = h + h
                    idxv[pl.ds((l * 8 + c) * B + v * L, L)] = h2
                    idxv[pl.ds(R + (l * 8 + c) * B + v * L, L)] = h2 + 1
            return c2

        lax.fori_loop(0, B // L, phase_a, 0)

    def fire(idxv, rA, rB, semA, semB):
        pltpu.async_copy(tab0.at[idxv.at[pl.ds(0, R)]], rA, semA)
        pltpu.async_copy(tab0.at[idxv.at[pl.ds(R, R)]], rB, semB)

    def drain(idxv, rA, rB, semA, semB):
        pltpu.make_async_copy(tab0.at[idxv.at[pl.ds(0, R)]], rA, semA).wait()
        pltpu.make_async_copy(tab0.at[idxv.at[pl.ds(R, R)]], rB, semB).wait()

    def interp_and_out(i, xyzv, rA, rB):
        base = wid * PW + i * B

        def phase_b(v, c2):
            pids3 = (v * L + iota) * 3
            x = plsc.load_gather(xyzv, [pids3])
            y = plsc.load_gather(xyzv, [pids3 + 1])
            z = plsc.load_gather(xyzv, [pids3 + 2])
            for l in range(N_LEVELS):
                res = RES[l]
                posx, posy, posz = x * res, y * res, z * res
                px = posx.astype(jnp.int32)
                py = posy.astype(jnp.int32)
                pz = posz.astype(jnp.int32)
                fx = posx - px.astype(jnp.float32)
                fy = posy - py.astype(jnp.float32)
                fz = posz - pz.astype(jnp.float32)
                wx = (1.0 - fx, fx)
                wy = (1.0 - fy, fy)
                wz = (1.0 - fz, fz)
                wyz = (wy[0] * wz[0], wy[0] * wz[1], wy[1] * wz[0], wy[1] * wz[1])
                acc0 = jnp.zeros((L,), jnp.float32)
                acc1 = jnp.zeros((L,), jnp.float32)
                for c in range(8):
                    bx, by, bz = (c >> 2) & 1, (c >> 1) & 1, c & 1
                    w = wx[bx] * wyz[by * 2 + bz]
                    s = (l * 8 + c) * B + v * L
                    acc0 = acc0 + rA[pl.ds(s, L)] * w
                    acc1 = acc1 + rB[pl.ds(s, L)] * w
                outv[2 * l, pl.ds(v * L, L)] = acc0
                outv[2 * l + 1, pl.ds(v * L, L)] = acc1
            return c2

        lax.fori_loop(0, B // L, phase_b, 0)
        pltpu.sync_copy(outv, enc.at[:, pl.ds(base, B)])

    stage_and_index(0, xyzv0, idx0)
    fire(idx0, r0e, r1e, s0e, s1e)

    def g_body(g, carry):
        i0 = 2 * g
        i1 = i0 + 1
        stage_and_index(i1, xyzv1, idx1)
        fire(idx1, r0o, r1o, s0o, s1o)
        drain(idx0, r0e, r1e, s0e, s1e)
        interp_and_out(i0, xyzv0, r0e, r1e)

        @pl.when(i1 + 1 < CH)
        def _():
            stage_and_index(i0 + 2, xyzv0, idx0)
            fire(idx0, r0e, r1e, s0e, s1e)

        drain(idx1, r0o, r1o, s0o, s1o)
        interp_and_out(i1, xyzv1, r0o, r1o)
        return carry

    lax.fori_loop(0, CH // 2, g_body, 0)


BN = 4096


def _mlp_body(encT_ref, w0, b0, w1, b1, w2, b2, out_ref):
    dn = (((0,), (0,)), ((), ()))
    h = jnp.maximum(
        lax.dot_general(encT_ref[...], w0[...], dn, preferred_element_type=jnp.float32)
        + b0[...],
        0.0,
    )
    h = jnp.maximum(
        jnp.dot(h, w1[...], preferred_element_type=jnp.float32) + b1[...], 0.0
    )
    out_ref[...] = jnp.dot(h, w2[...], preferred_element_type=jnp.float32) + b2[...]


def _mlp(encT, W0, b0, W1, b1, W2, b2):
    return pl.pallas_call(
        _mlp_body,
        grid=(N // BN,),
        in_specs=[
            pl.BlockSpec((IN_DIM, BN), lambda i: (0, i)),
            pl.BlockSpec((IN_DIM, WIDTH), lambda i: (0, 0)),
            pl.BlockSpec((1, WIDTH), lambda i: (0, 0)),
            pl.BlockSpec((WIDTH, WIDTH), lambda i: (0, 0)),
            pl.BlockSpec((1, WIDTH), lambda i: (0, 0)),
            pl.BlockSpec((WIDTH, OUT_DIM), lambda i: (0, 0)),
            pl.BlockSpec((1, OUT_DIM), lambda i: (0, 0)),
        ],
        out_specs=pl.BlockSpec((BN, OUT_DIM), lambda i: (i, 0)),
        out_shape=jax.ShapeDtypeStruct((N, OUT_DIM), jnp.float32),
    )(
        encT,
        W0,
        b0.reshape(1, WIDTH),
        W1,
        b1.reshape(1, WIDTH),
        W2,
        b2.reshape(1, OUT_DIM),
    )


def kernel(xyz, tables, W0, b0, W1, b1, W2, b2):
    tab = tables.reshape(N_LEVELS * TABLE_SIZE * F)
    encT = _encode(xyz.reshape(N * 3), tab)
    return _mlp(encT, W0, b0, W1, b1, W2, b2)


# speedup vs baseline: 1.6025x; 1.6025x over previous
import functools

import jax
import jax.numpy as jnp
import numpy as np
from jax import lax
from jax.experimental import pallas as pl
from jax.experimental.pallas import tpu as pltpu
from jax.experimental.pallas import tpu_sc as plsc

N = 524288
N_LEVELS = 16
F = 2
LOG2_T = 19
TABLE_SIZE = 1 << LOG2_T
BASE_RES = 16
SCALE = 1.38
WIDTH = 64
OUT_DIM = 16
IN_DIM = N_LEVELS * F

NC = 2
NS = 16
L = 16
NW = NC * NS
PW = N // NW
B = 64
CH = PW // B
RPP = N_LEVELS * 8
R = B * RPP

P1 = 2654435761 - (1 << 32)
P2 = 805459861
MASK = TABLE_SIZE - 1

RES = [float(np.floor(BASE_RES * SCALE**l)) for l in range(N_LEVELS)]

_mesh = plsc.VectorSubcoreMesh(core_axis_name="c", subcore_axis_name="s")


@functools.partial(
    pl.kernel,
    out_type=jax.ShapeDtypeStruct((IN_DIM, N), jnp.float32),
    mesh=_mesh,
    compiler_params=pltpu.CompilerParams(
        use_tc_tiling_on_sc=False, needs_layout_passes=False
    ),
    scratch_types=[
        pltpu.VMEM((B * 3,), jnp.float32),
        pltpu.VMEM((B * 3,), jnp.float32),
        pltpu.VMEM((2 * R,), jnp.int32),
        pltpu.VMEM((2 * R,), jnp.int32),
        pltpu.VMEM((R,), jnp.float32),
        pltpu.VMEM((R,), jnp.float32),
        pltpu.VMEM((R,), jnp.float32),
        pltpu.VMEM((R,), jnp.float32),
        pltpu.VMEM((IN_DIM, B), jnp.float32),
        pltpu.SemaphoreType.DMA,
        pltpu.SemaphoreType.DMA,
        pltpu.SemaphoreType.DMA,
        pltpu.SemaphoreType.DMA,
    ],
)
def _encode(
    xyz, tabA, tabB, enc,
    xyzv0, xyzv1, idx0, idx1, r0e, r0o, r1e, r1o, outv,
    s0e, s0o, s1e, s1o,
):
    wid = lax.axis_index("s") * NC + lax.axis_index("c")
    iota = lax.iota(jnp.int32, L)

    def stage_and_index(i, xyzv, idxv):
        base = wid * PW + i * B
        pltpu.sync_copy(xyz.at[pl.ds(base * 3, B * 3)], xyzv)

        def phase_a(v, c2):
            pids3 = (v * L + iota) * 3
            x = plsc.load_gather(xyzv, [pids3])
            y = plsc.load_gather(xyzv, [pids3 + 1])
            z = plsc.load_gather(xyzv, [pids3 + 2])
            for l in range(N_LEVELS):
                res = RES[l]
                px = (x * res).astype(jnp.int32)
                py = (y * res).astype(jnp.int32)
                pz = (z * res).astype(jnp.int32)
                loff = (l % 8) * TABLE_SIZE
                hyt = py * P1
                hy = (jnp.bitwise_and(hyt, MASK), jnp.bitwise_and(hyt + P1, MASK))
                hzt = pz * P2
                hz = (
                    jnp.bitwise_or(jnp.bitwise_and(hzt, MASK), loff),
                    jnp.bitwise_or(jnp.bitwise_and(hzt + P2, MASK), loff),
                )
                hxy = (px ^ hy[0], px ^ hy[1], (px + 1) ^ hy[0], (px + 1) ^ hy[1])
                for c in range(8):
                    bx, by, bz = (c >> 2) & 1, (c >> 1) & 1, c & 1
                    h = hxy[bx * 2 + by] ^ hz[bz]
                    h2 = h + h
                    idxv[pl.ds((l * 8 + c) * B + v * L, L)] = h2
                    idxv[pl.ds(R + (l * 8 + c) * B + v * L, L)] = h2 + 1
            return c2

        lax.fori_loop(0, B // L, phase_a, 0)

    H = R // 2

    def fire(idxv, rA, rB, semA, semB):
        pltpu.async_copy(tabA.at[idxv.at[pl.ds(0, H)]], rA.at[pl.ds(0, H)], semA)
        pltpu.async_copy(tabB.at[idxv.at[pl.ds(H, H)]], rA.at[pl.ds(H, H)], semA)
        pltpu.async_copy(tabA.at[idxv.at[pl.ds(R, H)]], rB.at[pl.ds(0, H)], semB)
        pltpu.async_copy(tabB.at[idxv.at[pl.ds(R + H, H)]], rB.at[pl.ds(H, H)], semB)

    def drain(idxv, rA, rB, semA, semB):
        pltpu.make_async_copy(tabA.at[idxv.at[pl.ds(0, H)]], rA.at[pl.ds(0, H)], semA).wait()
        pltpu.make_async_copy(tabB.at[idxv.at[pl.ds(H, H)]], rA.at[pl.ds(H, H)], semA).wait()
        pltpu.make_async_copy(tabA.at[idxv.at[pl.ds(R, H)]], rB.at[pl.ds(0, H)], semB).wait()
        pltpu.make_async_copy(tabB.at[idxv.at[pl.ds(R + H, H)]], rB.at[pl.ds(H, H)], semB).wait()

    def interp_and_out(i, xyzv, rA, rB):
        base = wid * PW + i * B

        def phase_b(v, c2):
            pids3 = (v * L + iota) * 3
            x = plsc.load_gather(xyzv, [pids3])
            y = plsc.load_gather(xyzv, [pids3 + 1])
            z = plsc.load_gather(xyzv, [pids3 + 2])
            for l in range(N_LEVELS):
                res = RES[l]
                posx, posy, posz = x * res, y * res, z * res
                px = posx.astype(jnp.int32)
                py = posy.astype(jnp.int32)
                pz = posz.astype(jnp.int32)
                fx = posx - px.astype(jnp.float32)
                fy = posy - py.astype(jnp.float32)
                fz = posz - pz.astype(jnp.float32)
                wx = (1.0 - fx, fx)
                wy = (1.0 - fy, fy)
                wz = (1.0 - fz, fz)
                wyz = (wy[0] * wz[0], wy[0] * wz[1], wy[1] * wz[0], wy[1] * wz[1])
                acc0 = jnp.zeros((L,), jnp.float32)
                acc1 = jnp.zeros((L,), jnp.float32)
                for c in range(8):
                    bx, by, bz = (c >> 2) & 1, (c >> 1) & 1, c & 1
                    w = wx[bx] * wyz[by * 2 + bz]
                    s = (l * 8 + c) * B + v * L
                    acc0 = acc0 + rA[pl.ds(s, L)] * w
                    acc1 = acc1 + rB[pl.ds(s, L)] * w
                outv[2 * l, pl.ds(v * L, L)] = acc0
                outv[2 * l + 1, pl.ds(v * L, L)] = acc1
            return c2

        lax.fori_loop(0, B // L, phase_b, 0)
        pltpu.sync_copy(outv, enc.at[:, pl.ds(base, B)])

    stage_and_index(0, xyzv0, idx0)
    fire(idx0, r0e, r1e, s0e, s1e)

    def g_body(g, carry):
        i0 = 2 * g
        i1 = i0 + 1
        stage_and_index(i1, xyzv1, idx1)
        fire(idx1, r0o, r1o, s0o, s1o)
        drain(idx0, r0e, r1e, s0e, s1e)
        interp_and_out(i0, xyzv0, r0e, r1e)

        @pl.when(i1 + 1 < CH)
        def _():
            stage_and_index(i0 + 2, xyzv0, idx0)
            fire(idx0, r0e, r1e, s0e, s1e)

        drain(idx1, r0o, r1o, s0o, s1o)
        interp_and_out(i1, xyzv1, r0o, r1o)
        return carry

    lax.fori_loop(0, CH // 2, g_body, 0)


BN = 4096


def _mlp_body(encT_ref, w0, b0, w1, b1, w2, b2, out_ref):
    dn = (((0,), (0,)), ((), ()))
    h = jnp.maximum(
        lax.dot_general(encT_ref[...], w0[...], dn, preferred_element_type=jnp.float32)
        + b0[...],
        0.0,
    )
    h = jnp.maximum(
        jnp.dot(h, w1[...], preferred_element_type=jnp.float32) + b1[...], 0.0
    )
    out_ref[...] = jnp.dot(h, w2[...], preferred_element_type=jnp.float32) + b2[...]


def _mlp(encT, W0, b0, W1, b1, W2, b2):
    return pl.pallas_call(
        _mlp_body,
        grid=(N // BN,),
        in_specs=[
            pl.BlockSpec((IN_DIM, BN), lambda i: (0, i)),
            pl.BlockSpec((IN_DIM, WIDTH), lambda i: (0, 0)),
            pl.BlockSpec((1, WIDTH), lambda i: (0, 0)),
            pl.BlockSpec((WIDTH, WIDTH), lambda i: (0, 0)),
            pl.BlockSpec((1, WIDTH), lambda i: (0, 0)),
            pl.BlockSpec((WIDTH, OUT_DIM), lambda i: (0, 0)),
            pl.BlockSpec((1, OUT_DIM), lambda i: (0, 0)),
        ],
        out_specs=pl.BlockSpec((BN, OUT_DIM), lambda i: (i, 0)),
        out_shape=jax.ShapeDtypeStruct((N, OUT_DIM), jnp.float32),
    )(
        encT,
        W0,
        b0.reshape(1, WIDTH),
        W1,
        b1.reshape(1, WIDTH),
        W2,
        b2.reshape(1, OUT_DIM),
    )


def kernel(xyz, tables, W0, b0, W1, b1, W2, b2):
    half = N_LEVELS // 2
    tabA = tables[:half].reshape(half * TABLE_SIZE * F)
    tabB = tables[half:].reshape(half * TABLE_SIZE * F)
    encT = _encode(xyz.reshape(N * 3), tabA, tabB)
    return _mlp(encT, W0, b0, W1, b1, W2, b2)


# speedup vs baseline: 4.3724x; 2.7285x over previous
import functools

import jax
import jax.numpy as jnp
import numpy as np
from jax import lax
from jax.experimental import pallas as pl
from jax.experimental.pallas import tpu as pltpu
from jax.experimental.pallas import tpu_sc as plsc

N = 524288
N_LEVELS = 16
F = 2
LOG2_T = 19
TABLE_SIZE = 1 << LOG2_T
BASE_RES = 16
SCALE = 1.38
WIDTH = 64
OUT_DIM = 16
IN_DIM = N_LEVELS * F

NC = 2
NS = 16
L = 16
NW = NC * NS
PW = N // NW
B = 32
CH = PW // B
RPP = N_LEVELS * 8
R = B * RPP
HALF = N_LEVELS // 2 * TABLE_SIZE

P1 = 2654435761 - (1 << 32)
P2 = 805459861
MASK = TABLE_SIZE - 1

RES = [float(np.floor(BASE_RES * SCALE**l)) for l in range(N_LEVELS)]

_mesh = plsc.VectorSubcoreMesh(core_axis_name="c", subcore_axis_name="s")

_sc_params = pltpu.CompilerParams(
    use_tc_tiling_on_sc=False, needs_layout_passes=False
)

PK = 4096
PC = HALF // NW // PK


@functools.partial(
    pl.kernel,
    out_type=(
        jax.ShapeDtypeStruct((HALF, F), jnp.float32),
        jax.ShapeDtypeStruct((HALF, F), jnp.float32),
    ),
    mesh=_mesh,
    compiler_params=_sc_params,
    scratch_types=[
        pltpu.VMEM((PK,), jnp.float32),
        pltpu.VMEM((PK,), jnp.float32),
        pltpu.VMEM((PK, F), jnp.float32),
    ],
)
def _interleave(tab0, tab1, pA, pB, f0v, f1v, scr):
    wid = lax.axis_index("s") * NC + lax.axis_index("c")
    iota = lax.iota(jnp.int32, L)
    zero = jnp.zeros((L,), jnp.int32)
    one = jnp.full((L,), 1, jnp.int32)

    def do_half(hoff, out_ref):
        def chunk(i, carry):
            row0 = hoff + (wid * PC + i) * PK
            pltpu.sync_copy(tab0.at[pl.ds(row0, PK)], f0v)
            pltpu.sync_copy(tab1.at[pl.ds(row0, PK)], f1v)

            def v_body(v, c2):
                pids = v * L + iota
                f0 = f0v[pl.ds(v * L, L)]
                f1 = f1v[pl.ds(v * L, L)]
                plsc.store_scatter(scr, [pids, zero], f0)
                plsc.store_scatter(scr, [pids, one], f1)
                return c2

            lax.fori_loop(0, PK // L, v_body, 0)
            pltpu.sync_copy(scr, out_ref.at[pl.ds(row0 - hoff, PK)])
            return carry

        lax.fori_loop(0, PC, chunk, 0)

    do_half(0, pA)
    do_half(HALF, pB)


@functools.partial(
    pl.kernel,
    out_type=jax.ShapeDtypeStruct((IN_DIM, N), jnp.float32),
    mesh=_mesh,
    compiler_params=_sc_params,
    scratch_types=[
        pltpu.VMEM((PW * 3,), jnp.float32),
        pltpu.VMEM((R,), jnp.int32),
        pltpu.VMEM((R,), jnp.int32),
        pltpu.VMEM((R, F), jnp.float32),
        pltpu.VMEM((R, F), jnp.float32),
        pltpu.VMEM((IN_DIM, B), jnp.float32),
        pltpu.SemaphoreType.DMA,
        pltpu.SemaphoreType.DMA,
    ],
)
def _encode(
    xyz, pA, pB, enc,
    xyzv, idx0, idx1, re_, ro_, outv,
    se, so,
):
    wid = lax.axis_index("s") * NC + lax.axis_index("c")
    iota = lax.iota(jnp.int32, L)
    zero = jnp.zeros((L,), jnp.int32)
    one = jnp.full((L,), 1, jnp.int32)
    H = R // 2

    pltpu.sync_copy(xyz.at[pl.ds(wid * PW * 3, PW * 3)], xyzv)

    def stage_and_index(i, idxv):
        def phase_a(v, c2):
            pids3 = (i * B + v * L + iota) * 3
            x = plsc.load_gather(xyzv, [pids3])
            y = plsc.load_gather(xyzv, [pids3 + 1])
            z = plsc.load_gather(xyzv, [pids3 + 2])
            for l in range(N_LEVELS):
                res = RES[l]
                px = (x * res).astype(jnp.int32)
                py = (y * res).astype(jnp.int32)
                pz = (z * res).astype(jnp.int32)
                loff = (l % 8) * TABLE_SIZE
                hyt = py * P1
                hy = (jnp.bitwise_and(hyt, MASK), jnp.bitwise_and(hyt + P1, MASK))
                hzt = pz * P2
                hz = (
                    jnp.bitwise_or(jnp.bitwise_and(hzt, MASK), loff),
                    jnp.bitwise_or(jnp.bitwise_and(hzt + P2, MASK), loff),
                )
                hxy = (px ^ hy[0], px ^ hy[1], (px + 1) ^ hy[0], (px + 1) ^ hy[1])
                for c in range(8):
                    bx, by, bz = (c >> 2) & 1, (c >> 1) & 1, c & 1
                    h = hxy[bx * 2 + by] ^ hz[bz]
                    idxv[pl.ds((l * 8 + c) * B + v * L, L)] = h
            return c2

        lax.fori_loop(0, B // L, phase_a, 0)

    def fire(idxv, rv, sem):
        pltpu.async_copy(pA.at[idxv.at[pl.ds(0, H)]], rv.at[pl.ds(0, H)], sem)
        pltpu.async_copy(pB.at[idxv.at[pl.ds(H, H)]], rv.at[pl.ds(H, H)], sem)

    def drain(idxv, rv, sem):
        pltpu.make_async_copy(pA.at[idxv.at[pl.ds(0, H)]], rv.at[pl.ds(0, H)], sem).wait()
        pltpu.make_async_copy(pB.at[idxv.at[pl.ds(H, H)]], rv.at[pl.ds(H, H)], sem).wait()

    def interp_and_out(i, rv):
        base = wid * PW + i * B

        def phase_b(v, c2):
            pids3 = (i * B + v * L + iota) * 3
            x = plsc.load_gather(xyzv, [pids3])
            y = plsc.load_gather(xyzv, [pids3 + 1])
            z = plsc.load_gather(xyzv, [pids3 + 2])
            pids = v * L + iota
            for l in range(N_LEVELS):
                res = RES[l]
                posx, posy, posz = x * res, y * res, z * res
                px = posx.astype(jnp.int32)
                py = posy.astype(jnp.int32)
                pz = posz.astype(jnp.int32)
                fx = posx - px.astype(jnp.float32)
                fy = posy - py.astype(jnp.float32)
                fz = posz - pz.astype(jnp.float32)
                wx = (1.0 - fx, fx)
                wy = (1.0 - fy, fy)
                wz = (1.0 - fz, fz)
                wyz = (wy[0] * wz[0], wy[0] * wz[1], wy[1] * wz[0], wy[1] * wz[1])
                acc0 = jnp.zeros((L,), jnp.float32)
                acc1 = jnp.zeros((L,), jnp.float32)
                for c in range(8):
                    bx, by, bz = (c >> 2) & 1, (c >> 1) & 1, c & 1
                    w = wx[bx] * wyz[by * 2 + bz]
                    rows = (l * 8 + c) * B + pids
                    f0 = plsc.load_gather(rv, [rows, zero])
                    f1 = plsc.load_gather(rv, [rows, one])
                    acc0 = acc0 + f0 * w
                    acc1 = acc1 + f1 * w
                outv[2 * l, pl.ds(v * L, L)] = acc0
                outv[2 * l + 1, pl.ds(v * L, L)] = acc1
            return c2

        lax.fori_loop(0, B // L, phase_b, 0)
        pltpu.sync_copy(outv, enc.at[:, pl.ds(base, B)])

    stage_and_index(0, idx0)
    fire(idx0, re_, se)

    def g_body(g, carry):
        i0 = 2 * g
        i1 = i0 + 1
        stage_and_index(i1, idx1)
        fire(idx1, ro_, so)
        drain(idx0, re_, se)
        interp_and_out(i0, re_)

        @pl.when(i1 + 1 < CH)
        def _():
            stage_and_index(i0 + 2, idx0)
            fire(idx0, re_, se)

        drain(idx1, ro_, so)
        interp_and_out(i1, ro_)
        return carry

    lax.fori_loop(0, CH // 2, g_body, 0)


BN = 4096


def _mlp_body(encT_ref, w0, b0, w1, b1, w2, b2, out_ref):
    dn = (((0,), (0,)), ((), ()))
    h = jnp.maximum(
        lax.dot_general(encT_ref[...], w0[...], dn, preferred_element_type=jnp.float32)
        + b0[...],
        0.0,
    )
    h = jnp.maximum(
        jnp.dot(h, w1[...], preferred_element_type=jnp.float32) + b1[...], 0.0
    )
    out_ref[...] = jnp.dot(h, w2[...], preferred_element_type=jnp.float32) + b2[...]


def _mlp(encT, W0, b0, W1, b1, W2, b2):
    return pl.pallas_call(
        _mlp_body,
        grid=(N // BN,),
        in_specs=[
            pl.BlockSpec((IN_DIM, BN), lambda i: (0, i)),
            pl.BlockSpec((IN_DIM, WIDTH), lambda i: (0, 0)),
            pl.BlockSpec((1, WIDTH), lambda i: (0, 0)),
            pl.BlockSpec((WIDTH, WIDTH), lambda i: (0, 0)),
            pl.BlockSpec((1, WIDTH), lambda i: (0, 0)),
            pl.BlockSpec((WIDTH, OUT_DIM), lambda i: (0, 0)),
            pl.BlockSpec((1, OUT_DIM), lambda i: (0, 0)),
        ],
        out_specs=pl.BlockSpec((BN, OUT_DIM), lambda i: (i, 0)),
        out_shape=jax.ShapeDtypeStruct((N, OUT_DIM), jnp.float32),
    )(
        encT,
        W0,
        b0.reshape(1, WIDTH),
        W1,
        b1.reshape(1, WIDTH),
        W2,
        b2.reshape(1, OUT_DIM),
    )


def kernel(xyz, tables, W0, b0, W1, b1, W2, b2):
    tab0 = tables[:, :, 0].reshape(N_LEVELS * TABLE_SIZE)
    tab1 = tables[:, :, 1].reshape(N_LEVELS * TABLE_SIZE)
    pA, pB = _interleave(tab0, tab1)
    encT = _encode(xyz.reshape(N * 3), pA, pB)
    return _mlp(encT, W0, b0, W1, b1, W2, b2)
